# Initial kernel scaffold; baseline (speedup 1.0000x reference)
import functools

import jax
import jax.numpy as jnp
from jax import lax
from jax.experimental import pallas as pl
from jax.experimental.pallas import tpu as pltpu
from jax.experimental.pallas import tpu_sc as plsc

B = 16384
W = 10
NT = 5
D = 32
NW = 32
PB = B // NW
CH = 64
NCH = PB // CH
IW = 64
NGC = CH * W // IW
NGT = CH * NT // IW


def _sc_body(ctx_idx_hbm, tgt_idx_hbm, ctx_table, tgt_table, out_hbm,
             ctx_idx_v, tgt_idx_v, ctx_rows, tgt_rows, out_v,
             isem, gsem0, gsem1, osem):
    wid = lax.axis_index("s") * 2 + lax.axis_index("c")
    cbase = wid * (PB * W // IW)
    tbase = wid * (PB * NT // IW)
    lane = lax.iota(jnp.int32, 16)
    gsems = (gsem0, gsem1)

    def fire_idx(c):
        p = c % 2
        a = pltpu.async_copy(
            ctx_idx_hbm.at[pl.ds(cbase + c * NGC, NGC)], ctx_idx_v.at[p], isem)
        b = pltpu.async_copy(
            tgt_idx_hbm.at[pl.ds(tbase + c * NGT, NGT)], tgt_idx_v.at[p], isem)
        return [a, b]

    def fire_gathers(c):
        p = c % 2
        ds = []
        for j in range(NGC):
            ds.append(pltpu.async_copy(
                ctx_table.at[ctx_idx_v.at[p, j]],
                ctx_rows.at[p, pl.ds(j * IW, IW), :], gsems[p]))
        for j in range(NGT):
            ds.append(pltpu.async_copy(
                tgt_table.at[tgt_idx_v.at[p, j]],
                tgt_rows.at[p, pl.ds(j * IW, IW), :], gsems[p]))
        return ds

    def compute(c):
        p = c % 2
        rows = ctx_rows.at[p]
        trows = tgt_rows.at[p]
        ov = out_v.at[p]

        def body(b, carry):
            b10 = b * W
            b5 = b * NT
            acc0 = rows[b10, pl.ds(0, 16)]
            acc1 = rows[b10, pl.ds(16, 16)]
            for w in range(1, W):
                acc0 = acc0 + rows[b10 + w, pl.ds(0, 16)]
                acc1 = acc1 + rows[b10 + w, pl.ds(16, 16)]
            outv = jnp.zeros((16,), jnp.float32)
            for n in range(NT):
                t0 = trows[b5 + n, pl.ds(0, 16)]
                t1 = trows[b5 + n, pl.ds(16, 16)]
                d = jnp.sum(t0 * acc0 + t1 * acc1)
                outv = jnp.where(lane == n, d, outv)
            plsc.store_scatter(ov, [b5 + lane], outv, mask=lane < NT)
            return carry

        lax.fori_loop(0, CH, body, 0)

    for d in fire_idx(0):
        d.wait()
    pending_g = fire_gathers(0)
    pending_i = fire_idx(1)
    pending_o = []

    for c in range(NCH):
        if c + 1 < NCH:
            for d in pending_i:
                d.wait()
            next_g = fire_gathers(c + 1)
        else:
            next_g = []
        for d in pending_g:
            d.wait()
        if c + 2 < NCH:
            pending_i = fire_idx(c + 2)
        if len(pending_o) >= 2:
            pending_o.pop(0).wait()
        compute(c)
        off = wid * (PB * NT) + c * (CH * NT)
        pending_o.append(pltpu.async_copy(
            out_v.at[c % 2], out_hbm.at[pl.ds(off, CH * NT)], osem))
        pending_g = next_g

    for d in pending_o:
        d.wait()


@jax.jit
def _cbow_ns(ctx_idx, tgt_idx, ctx_table, tgt_table):
    mesh = plsc.VectorSubcoreMesh(core_axis_name="c", subcore_axis_name="s")
    return pl.kernel(
        _sc_body,
        out_type=jax.ShapeDtypeStruct((B * NT,), jnp.float32),
        mesh=mesh,
        scratch_types=[
            pltpu.VMEM((2, NGC, IW), jnp.int32),
            pltpu.VMEM((2, NGT, IW), jnp.int32),
            pltpu.VMEM((2, CH * W, D), jnp.float32),
            pltpu.VMEM((2, CH * NT, D), jnp.float32),
            pltpu.VMEM((2, CH * NT), jnp.float32),
            pltpu.SemaphoreType.DMA,
            pltpu.SemaphoreType.DMA,
            pltpu.SemaphoreType.DMA,
            pltpu.SemaphoreType.DMA,
        ],
    )(ctx_idx, tgt_idx, ctx_table, tgt_table)


def kernel(context, target, ctx_table, tgt_table):
    ctx_idx = context.astype(jnp.int32).reshape(-1, IW)
    tgt_idx = target.astype(jnp.int32).reshape(-1, IW)
    out = _cbow_ns(ctx_idx, tgt_idx, ctx_table, tgt_table)
    return out.reshape(B, NT)


# speedup vs baseline: 2.2953x; 2.2953x over previous
import functools

import jax
import jax.numpy as jnp
from jax import lax
from jax.experimental import pallas as pl
from jax.experimental.pallas import tpu as pltpu
from jax.experimental.pallas import tpu_sc as plsc

B = 16384
W = 10
NT = 5
D = 32
NW = 32
PB = B // NW
CH = 64
NCH = PB // CH
IW = 64
NGC = CH * W // IW
NGT = CH * NT // IW


def _sc_body(ctx_idx_hbm, tgt_idx_hbm, ctx_table, tgt_table, out_hbm,
             ctx_idx0, ctx_idx1, tgt_idx0, tgt_idx1,
             ctx_rows0, ctx_rows1, tgt_rows0, tgt_rows1, p_v,
             out_v0, out_v1, isem, gsem0, gsem1, osem):
    ctx_idxs = (ctx_idx0, ctx_idx1)
    tgt_idxs = (tgt_idx0, tgt_idx1)
    ctx_rowss = (ctx_rows0, ctx_rows1)
    tgt_rowss = (tgt_rows0, tgt_rows1)
    out_vs = (out_v0, out_v1)
    wid = lax.axis_index("s") * 2 + lax.axis_index("c")
    cbase = wid * (PB * W)
    tbase = wid * (PB * NT)
    lane = lax.iota(jnp.int32, 16)
    gsems = (gsem0, gsem1)

    def fire_idx(c):
        p = c % 2
        a = pltpu.async_copy(
            ctx_idx_hbm.at[pl.ds(cbase + c * CH * W, CH * W)],
            ctx_idxs[p], isem)
        b = pltpu.async_copy(
            tgt_idx_hbm.at[pl.ds(tbase + c * CH * NT, CH * NT)],
            tgt_idxs[p], isem)
        return [a, b]

    def fire_gathers(c):
        p = c % 2
        ds = []
        for j in range(NGC):
            ds.append(pltpu.async_copy(
                ctx_table.at[ctx_idxs[p].at[pl.ds(j * IW, IW)]],
                ctx_rowss[p].at[pl.ds(j * IW, IW), :], gsems[p]))
        for j in range(NGT):
            ds.append(pltpu.async_copy(
                tgt_table.at[tgt_idxs[p].at[pl.ds(j * IW, IW)]],
                tgt_rowss[p].at[pl.ds(j * IW, IW), :], gsems[p]))
        return ds

    def compute(c):
        p = c % 2
        rows = ctx_rowss[p]
        trows = tgt_rowss[p]
        ov = out_vs[p]

        def sum_body(b, carry):
            b10 = b * W
            acc0 = rows[b10, pl.ds(0, 16)]
            acc1 = rows[b10, pl.ds(16, 16)]
            for w in range(1, W):
                acc0 = acc0 + rows[b10 + w, pl.ds(0, 16)]
                acc1 = acc1 + rows[b10 + w, pl.ds(16, 16)]
            b5 = b * NT
            for n in range(NT):
                t0 = trows[b5 + n, pl.ds(0, 16)]
                t1 = trows[b5 + n, pl.ds(16, 16)]
                q32 = (b5 + n) * D
                p_v[pl.ds(q32, 16)] = t0 * acc0
                p_v[pl.ds(q32 + 16, 16)] = t1 * acc1
            return carry

        lax.fori_loop(0, CH, sum_body, 0)

        def dot_body(g, carry):
            base = (g * 16 + lane) * D
            a0 = jnp.zeros((16,), jnp.float32)
            a1 = jnp.zeros((16,), jnp.float32)
            a2 = jnp.zeros((16,), jnp.float32)
            a3 = jnp.zeros((16,), jnp.float32)
            for k in range(0, D, 4):
                a0 = a0 + plsc.load_gather(p_v, [base + k])
                a1 = a1 + plsc.load_gather(p_v, [base + (k + 1)])
                a2 = a2 + plsc.load_gather(p_v, [base + (k + 2)])
                a3 = a3 + plsc.load_gather(p_v, [base + (k + 3)])
            ov[pl.ds(g * 16, 16)] = (a0 + a1) + (a2 + a3)
            return carry

        lax.fori_loop(0, CH * NT // 16, dot_body, 0)

    for d in fire_idx(0):
        d.wait()
    pending_g = fire_gathers(0)
    pending_i = fire_idx(1)
    pending_o = []

    for c in range(NCH):
        if c + 1 < NCH:
            for d in pending_i:
                d.wait()
            next_g = fire_gathers(c + 1)
        else:
            next_g = []
        for d in pending_g:
            d.wait()
        if c + 2 < NCH:
            pending_i = fire_idx(c + 2)
        if len(pending_o) >= 2:
            pending_o.pop(0).wait()
        compute(c)
        off = wid * (PB * NT) + c * (CH * NT)
        pending_o.append(pltpu.async_copy(
            out_vs[c % 2], out_hbm.at[pl.ds(off, CH * NT)], osem))
        pending_g = next_g

    for d in pending_o:
        d.wait()


@jax.jit
def _cbow_ns(ctx_idx, tgt_idx, ctx_table, tgt_table):
    mesh = plsc.VectorSubcoreMesh(core_axis_name="c", subcore_axis_name="s")
    return pl.kernel(
        _sc_body,
        out_type=jax.ShapeDtypeStruct((B * NT,), jnp.float32),
        mesh=mesh,
        compiler_params=pltpu.CompilerParams(
            needs_layout_passes=False, use_tc_tiling_on_sc=False),
        scratch_types=[
            pltpu.VMEM((CH * W,), jnp.int32),
            pltpu.VMEM((CH * W,), jnp.int32),
            pltpu.VMEM((CH * NT,), jnp.int32),
            pltpu.VMEM((CH * NT,), jnp.int32),
            pltpu.VMEM((CH * W, D), jnp.float32),
            pltpu.VMEM((CH * W, D), jnp.float32),
            pltpu.VMEM((CH * NT, D), jnp.float32),
            pltpu.VMEM((CH * NT, D), jnp.float32),
            pltpu.VMEM((CH * NT * D,), jnp.float32),
            pltpu.VMEM((CH * NT,), jnp.float32),
            pltpu.VMEM((CH * NT,), jnp.float32),
            pltpu.SemaphoreType.DMA,
            pltpu.SemaphoreType.DMA,
            pltpu.SemaphoreType.DMA,
            pltpu.SemaphoreType.DMA,
        ],
    )(ctx_idx, tgt_idx, ctx_table, tgt_table)


def kernel(context, target, ctx_table, tgt_table):
    ctx_idx = context.astype(jnp.int32).reshape(-1)
    tgt_idx = target.astype(jnp.int32).reshape(-1)
    out = _cbow_ns(ctx_idx, tgt_idx, ctx_table, tgt_table)
    return out.reshape(B, NT)
